# baseline (speedup 1.0000x reference)
import functools

import jax
import jax.numpy as jnp
from jax import lax
from jax.experimental import pallas as pl
from jax.experimental.pallas import tpu as pltpu
from jax.experimental.pallas import tpu_sc as plsc

D = 128
NC = 2
NS = 16
NW = NC * NS
CH = 128
NBUF = 2
SB = 16
DUMP_PAD = 240

_mesh = plsc.VectorSubcoreMesh(core_axis_name="c", subcore_axis_name="s")


def _seg_sum_body(n_rows, k, x_hbm, src_hbm, dst_hbm, zeros_hbm, out_hbm,
                  src_v, dst_v, rows0, rows1, acc, gsem0, gsem1):
    cid = lax.axis_index("c")
    sid = lax.axis_index("s")
    wid = cid * NS + sid
    n_acc = n_rows + DUMP_PAD
    acc_slab = n_acc // NS
    pltpu.sync_copy(zeros_hbm.at[pl.ds(sid * acc_slab, acc_slab)],
                    acc.at[pl.ds(sid * acc_slab, acc_slab)])
    plsc.subcore_barrier()

    for h in range(2):
        pltpu.sync_copy(src_hbm.at[2 * wid + h], src_v)
        pltpu.sync_copy(dst_hbm.at[2 * wid + h], dst_v)

        def rnd(r, carry):
            j0 = 2 * r
            g0 = pltpu.async_copy(x_hbm.at[src_v.at[j0]], rows0, gsem0)
            g1 = pltpu.async_copy(x_hbm.at[src_v.at[j0 + 1]], rows1, gsem1)
            g0.wait()
            pltpu.sync_copy(rows0, acc.at[dst_v.at[j0]], add=True)
            g1.wait()
            pltpu.sync_copy(rows1, acc.at[dst_v.at[j0 + 1]], add=True)
            return carry

        lax.fori_loop(0, k // 4, rnd, 0)
    plsc.subcore_barrier()
    pltpu.sync_copy(acc.at[pl.ds(sid * acc_slab, acc_slab)],
                    out_hbm.at[cid, pl.ds(sid * acc_slab, acc_slab)])


def _count_body(n_rows, k, dst_hbm, zeros_hbm, out_hbm,
                dst_v, ones_v, acc, sem):
    cid = lax.axis_index("c")
    sid = lax.axis_index("s")
    wid = cid * NS + sid
    n_acc = n_rows + DUMP_PAD
    acc_slab = n_acc // NS
    pltpu.sync_copy(dst_hbm.at[wid], dst_v)
    ones16 = jnp.ones((16,), jnp.float32)

    def obody(r, carry):
        for g in range(D // 16):
            ones_v[r, pl.ds(g * 16, 16)] = ones16
        return carry

    lax.fori_loop(0, CH, obody, 0)
    pltpu.sync_copy(zeros_hbm.at[pl.ds(sid * acc_slab, acc_slab)],
                    acc.at[pl.ds(sid * acc_slab, acc_slab)])
    plsc.subcore_barrier()

    del sem

    def body(j, carry):
        pltpu.sync_copy(ones_v, acc.at[dst_v.at[j]], add=True)
        return carry

    lax.fori_loop(0, k, body, 0)
    plsc.subcore_barrier()
    pltpu.sync_copy(acc.at[pl.ds(sid * acc_slab, acc_slab)],
                    out_hbm.at[cid, pl.ds(sid * acc_slab, acc_slab)])


@functools.lru_cache(maxsize=None)
def _make_seg_sum(n_rows, k):
    return pl.kernel(
        functools.partial(_seg_sum_body, n_rows, k),
        out_type=jax.ShapeDtypeStruct((NC, n_rows + DUMP_PAD, D), jnp.float32),
        mesh=_mesh,
        scratch_types=[
            pltpu.VMEM((k // 2, CH), jnp.int32),
            pltpu.VMEM((k // 2, CH), jnp.int32),
            pltpu.VMEM((CH, D), jnp.float32),
            pltpu.VMEM((CH, D), jnp.float32),
            pltpu.VMEM_SHARED((n_rows + DUMP_PAD, D), jnp.float32),
            pltpu.SemaphoreType.DMA,
            pltpu.SemaphoreType.DMA,
        ],
    )


@functools.lru_cache(maxsize=None)
def _make_count(n_rows, k):
    return pl.kernel(
        functools.partial(_count_body, n_rows, k),
        out_type=jax.ShapeDtypeStruct((NC, n_rows + DUMP_PAD, D), jnp.float32),
        mesh=_mesh,
        scratch_types=[
            pltpu.VMEM((k, CH), jnp.int32),
            pltpu.VMEM((CH, D), jnp.float32),
            pltpu.VMEM_SHARED((n_rows + DUMP_PAD, D), jnp.float32),
            pltpu.SemaphoreType.DMA,
        ],
    )


def _tc_update_body(agg_ref, cnt_ref, h_ref, w1_ref, w2_ref, o_ref):
    a = agg_ref[0] + agg_ref[1]
    c = cnt_ref[0][:, 0:1] + cnt_ref[1][:, 0:1]
    a = a / jnp.maximum(c, 1.0)
    o_ref[...] = jnp.maximum(
        jnp.dot(a, w1_ref[...], preferred_element_type=jnp.float32,
                precision=lax.Precision.HIGHEST)
        + jnp.dot(h_ref[...], w2_ref[...], preferred_element_type=jnp.float32,
                  precision=lax.Precision.HIGHEST),
        0.0,
    )


def _tc_update(agg, cnt, h, w1, w2):
    n = h.shape[0]
    b = 1000
    return pl.pallas_call(
        _tc_update_body,
        grid=(n // b,),
        in_specs=[
            pl.BlockSpec((NC, b, D), lambda i: (0, i, 0)),
            pl.BlockSpec((NC, b, D), lambda i: (0, i, 0)),
            pl.BlockSpec((b, D), lambda i: (i, 0)),
            pl.BlockSpec((D, D), lambda i: (0, 0)),
            pl.BlockSpec((D, D), lambda i: (0, 0)),
        ],
        out_specs=pl.BlockSpec((b, D), lambda i: (i, 0)),
        out_shape=jax.ShapeDtypeStruct((n, D), jnp.float32),
    )(agg, cnt, h, w1, w2)


def _tc_tanh_body(x_ref, w_ref, o_ref):
    o_ref[...] = jnp.tanh(
        jnp.dot(x_ref[...], w_ref[...], preferred_element_type=jnp.float32,
                precision=lax.Precision.HIGHEST))


def _tc_tanh(x, w):
    n = x.shape[0]
    b = 1000
    return pl.pallas_call(
        _tc_tanh_body,
        grid=(n // b,),
        in_specs=[
            pl.BlockSpec((b, D), lambda i: (i, 0)),
            pl.BlockSpec((D, D), lambda i: (0, 0)),
        ],
        out_specs=pl.BlockSpec((b, D), lambda i: (i, 0)),
        out_shape=jax.ShapeDtypeStruct((n, D), jnp.float32),
    )(x, w)


def kernel(d_feat, p_feat, dd_edge_index, dp_edge_index,
           Wd_att, Wp_att, W_dd, W_dd_self, W_dp, W_p_self, W_pd, W_d_self):
    n_drug = d_feat.shape[0]
    n_prot = p_feat.shape[0]
    e = dd_edge_index.shape[1]
    k = -(-e // (NW * CH))
    k = -(-k // 4) * 4
    e_pad = NW * k * CH

    def prep(idx, fill, spread=1):
        idx = idx.astype(jnp.int32)
        pad = fill + jnp.arange(e_pad - e, dtype=jnp.int32) % spread
        return jnp.concatenate([idx, pad]).reshape(NW, k, CH)

    def vk(a):
        return a

    def v3(a):
        return a.reshape(NW * 2, k // 2, CH)

    src_dd_g = prep(dd_edge_index[0], 0)
    dst_dd_s = prep(dd_edge_index[1], n_drug, DUMP_PAD)
    src_dp_g = prep(dp_edge_index[0], 0)
    src_dp_s = prep(dp_edge_index[0], n_drug, DUMP_PAD)
    dst_dp_g = prep(dp_edge_index[1], 0)
    dst_dp_s = prep(dp_edge_index[1], n_prot, DUMP_PAD)

    zeros_acc = jnp.zeros((n_drug + DUMP_PAD, D), jnp.float32)

    seg_sum = _make_seg_sum(n_drug, k)
    count = _make_count(n_drug, k)

    cnt_dd = count(vk(dst_dd_s), zeros_acc)
    cnt_p = count(vk(dst_dp_s), zeros_acc)
    cnt_d = count(vk(src_dp_s), zeros_acc)

    d_att = _tc_tanh(d_feat, Wd_att)
    p_att = _tc_tanh(p_feat, Wp_att)
    d, p = d_att, p_att
    for _ in range(2):
        s_dd = seg_sum(d, v3(src_dd_g), v3(dst_dd_s), zeros_acc)
        d = _tc_update(s_dd, cnt_dd, d, W_dd, W_dd_self)
        s_pd = seg_sum(p, v3(dst_dp_g), v3(src_dp_s), zeros_acc)
        s_dp = seg_sum(d, v3(src_dp_g), v3(dst_dp_s), zeros_acc)
        p_new = _tc_update(s_dp, cnt_p, p, W_dp, W_p_self)
        d = _tc_update(s_pd, cnt_d, d, W_pd, W_d_self)
        p = p_new
    return jnp.concatenate([d, p, d_att, p_att], axis=0)


# speedup vs baseline: 2.7514x; 2.7514x over previous
import functools

import jax
import jax.numpy as jnp
from jax import lax
from jax.experimental import pallas as pl
from jax.experimental.pallas import tpu as pltpu
from jax.experimental.pallas import tpu_sc as plsc

D = 128
NC = 2
NS = 16
NW = NC * NS
CH = 128
NBUF = 2
SB = 16
DUMP_PAD = 240

_mesh = plsc.VectorSubcoreMesh(core_axis_name="c", subcore_axis_name="s")


def _seg_sum_body(n_rows, k, x_hbm, src_hbm, dst_hbm, zeros_hbm, out_hbm,
                  src_v, dst_v, rows0, rows1, acc, gsem0, gsem1):
    cid = lax.axis_index("c")
    sid = lax.axis_index("s")
    wid = cid * NS + sid
    n_acc = n_rows + DUMP_PAD
    acc_slab = n_acc // NS
    pltpu.sync_copy(zeros_hbm.at[pl.ds(sid * acc_slab, acc_slab)],
                    acc.at[pl.ds(sid * acc_slab, acc_slab)])
    plsc.subcore_barrier()

    for h in range(2):
        pltpu.sync_copy(src_hbm.at[2 * wid + h], src_v)
        pltpu.sync_copy(dst_hbm.at[2 * wid + h], dst_v)

        def rnd(r, carry):
            j0 = 2 * r
            g0 = pltpu.async_copy(x_hbm.at[src_v.at[j0]], rows0, gsem0)
            g1 = pltpu.async_copy(x_hbm.at[src_v.at[j0 + 1]], rows1, gsem1)
            g0.wait()
            pltpu.sync_copy(rows0, acc.at[dst_v.at[j0]], add=True)
            g1.wait()
            pltpu.sync_copy(rows1, acc.at[dst_v.at[j0 + 1]], add=True)
            return carry

        lax.fori_loop(0, k // 4, rnd, 0)
    plsc.subcore_barrier()
    pltpu.sync_copy(acc.at[pl.ds(sid * acc_slab, acc_slab)],
                    out_hbm.at[cid, pl.ds(sid * acc_slab, acc_slab)])


def _count_body(n_rows, k, dst_hbm, zeros_hbm, out_hbm,
                dst_v, ones_v, acc, sem):
    cid = lax.axis_index("c")
    sid = lax.axis_index("s")
    wid = cid * NS + sid
    n_acc = n_rows + DUMP_PAD
    acc_slab = n_acc // NS
    pltpu.sync_copy(dst_hbm.at[wid], dst_v)
    ones16 = jnp.ones((16,), jnp.float32)

    def obody(r, carry):
        for g in range(D // 16):
            ones_v[r, pl.ds(g * 16, 16)] = ones16
        return carry

    lax.fori_loop(0, CH, obody, 0)
    pltpu.sync_copy(zeros_hbm.at[pl.ds(sid * acc_slab, acc_slab)],
                    acc.at[pl.ds(sid * acc_slab, acc_slab)])
    plsc.subcore_barrier()

    del sem

    def body(j, carry):
        pltpu.sync_copy(ones_v, acc.at[dst_v.at[j]], add=True)
        return carry

    lax.fori_loop(0, k, body, 0)
    plsc.subcore_barrier()
    pltpu.sync_copy(acc.at[pl.ds(sid * acc_slab, acc_slab)],
                    out_hbm.at[cid, pl.ds(sid * acc_slab, acc_slab)])


@functools.lru_cache(maxsize=None)
def _make_seg_sum(n_rows, k):
    return pl.kernel(
        functools.partial(_seg_sum_body, n_rows, k),
        out_type=jax.ShapeDtypeStruct((NC, n_rows + DUMP_PAD, D), jnp.float32),
        mesh=_mesh,
        scratch_types=[
            pltpu.VMEM((k // 2, CH), jnp.int32),
            pltpu.VMEM((k // 2, CH), jnp.int32),
            pltpu.VMEM((CH, D), jnp.float32),
            pltpu.VMEM((CH, D), jnp.float32),
            pltpu.VMEM_SHARED((n_rows + DUMP_PAD, D), jnp.float32),
            pltpu.SemaphoreType.DMA,
            pltpu.SemaphoreType.DMA,
        ],
    )


@functools.lru_cache(maxsize=None)
def _make_count(n_rows, k):
    return pl.kernel(
        functools.partial(_count_body, n_rows, k),
        out_type=jax.ShapeDtypeStruct((NC, n_rows + DUMP_PAD, D), jnp.float32),
        mesh=_mesh,
        scratch_types=[
            pltpu.VMEM((k, CH), jnp.int32),
            pltpu.VMEM((CH, D), jnp.float32),
            pltpu.VMEM_SHARED((n_rows + DUMP_PAD, D), jnp.float32),
            pltpu.SemaphoreType.DMA,
        ],
    )


def _tc_update_body(agg_ref, cnt_ref, h_ref, w1_ref, w2_ref, o_ref):
    a = agg_ref[0] + agg_ref[1]
    c = cnt_ref[0][:, 0:1] + cnt_ref[1][:, 0:1]
    a = a / jnp.maximum(c, 1.0)
    o_ref[...] = jnp.maximum(
        jnp.dot(a, w1_ref[...], preferred_element_type=jnp.float32,
                precision=lax.Precision.HIGHEST)
        + jnp.dot(h_ref[...], w2_ref[...], preferred_element_type=jnp.float32,
                  precision=lax.Precision.HIGHEST),
        0.0,
    )


def _tc_update(agg, cnt, h, w1, w2):
    n = h.shape[0]
    b = 1000
    return pl.pallas_call(
        _tc_update_body,
        grid=(n // b,),
        in_specs=[
            pl.BlockSpec((NC, b, D), lambda i: (0, i, 0)),
            pl.BlockSpec((NC, b, D), lambda i: (0, i, 0)),
            pl.BlockSpec((b, D), lambda i: (i, 0)),
            pl.BlockSpec((D, D), lambda i: (0, 0)),
            pl.BlockSpec((D, D), lambda i: (0, 0)),
        ],
        out_specs=pl.BlockSpec((b, D), lambda i: (i, 0)),
        out_shape=jax.ShapeDtypeStruct((n, D), jnp.float32),
    )(agg, cnt, h, w1, w2)


def _tc_tanh_body(x_ref, w_ref, o_ref):
    o_ref[...] = jnp.tanh(
        jnp.dot(x_ref[...], w_ref[...], preferred_element_type=jnp.float32,
                precision=lax.Precision.HIGHEST))


def _tc_tanh(x, w):
    n = x.shape[0]
    b = 1000
    return pl.pallas_call(
        _tc_tanh_body,
        grid=(n // b,),
        in_specs=[
            pl.BlockSpec((b, D), lambda i: (i, 0)),
            pl.BlockSpec((D, D), lambda i: (0, 0)),
        ],
        out_specs=pl.BlockSpec((b, D), lambda i: (i, 0)),
        out_shape=jax.ShapeDtypeStruct((n, D), jnp.float32),
    )(x, w)


def kernel(d_feat, p_feat, dd_edge_index, dp_edge_index,
           Wd_att, Wp_att, W_dd, W_dd_self, W_dp, W_p_self, W_pd, W_d_self):
    n_drug = d_feat.shape[0]
    n_prot = p_feat.shape[0]
    e = dd_edge_index.shape[1]
    k = -(-e // (NW * CH))
    k = -(-k // 4) * 4
    e_pad = NW * k * CH

    def prep(idx, fill, spread=1):
        idx = idx.astype(jnp.int32)
        pad = fill + jnp.arange(e_pad - e, dtype=jnp.int32) % spread
        return jnp.concatenate([idx, pad]).reshape(NW, k, CH)

    def vk(a):
        return a

    def v3(a):
        return a.reshape(NW * 2, k // 2, CH)

    src_dd_g = prep(dd_edge_index[0], 0, n_drug)
    dst_dd_s = prep(dd_edge_index[1], n_drug, DUMP_PAD)
    src_dp_g = prep(dp_edge_index[0], 0, n_drug)
    src_dp_s = prep(dp_edge_index[0], n_drug, DUMP_PAD)
    dst_dp_g = prep(dp_edge_index[1], 0, n_prot)
    dst_dp_s = prep(dp_edge_index[1], n_prot, DUMP_PAD)

    zeros_acc = jnp.zeros((n_drug + DUMP_PAD, D), jnp.float32)

    seg_sum = _make_seg_sum(n_drug, k)
    count = _make_count(n_drug, k)

    cnt_dd = count(vk(dst_dd_s), zeros_acc)
    cnt_p = count(vk(dst_dp_s), zeros_acc)
    cnt_d = count(vk(src_dp_s), zeros_acc)

    d_att = _tc_tanh(d_feat, Wd_att)
    p_att = _tc_tanh(p_feat, Wp_att)
    d, p = d_att, p_att
    for _ in range(2):
        s_dd = seg_sum(d, v3(src_dd_g), v3(dst_dd_s), zeros_acc)
        d = _tc_update(s_dd, cnt_dd, d, W_dd, W_dd_self)
        s_pd = seg_sum(p, v3(dst_dp_g), v3(src_dp_s), zeros_acc)
        s_dp = seg_sum(d, v3(src_dp_g), v3(dst_dp_s), zeros_acc)
        p_new = _tc_update(s_dp, cnt_p, p, W_dp, W_p_self)
        d = _tc_update(s_pd, cnt_d, d, W_pd, W_d_self)
        p = p_new
    return jnp.concatenate([d, p, d_att, p_att], axis=0)


# speedup vs baseline: 2.7517x; 1.0001x over previous
import functools

import jax
import jax.numpy as jnp
from jax import lax
from jax.experimental import pallas as pl
from jax.experimental.pallas import tpu as pltpu
from jax.experimental.pallas import tpu_sc as plsc

D = 128
NC = 2
NS = 16
NW = NC * NS
CH = 128
NBUF = 2
SB = 16
DUMP_PAD = 240

_mesh = plsc.VectorSubcoreMesh(core_axis_name="c", subcore_axis_name="s")


def _seg_sum_body(n_rows, k, x_hbm, src_hbm, dst_hbm, zeros_hbm, out_hbm,
                  src_v, dst_v, rows0, rows1, acc, gsem0, gsem1):
    cid = lax.axis_index("c")
    sid = lax.axis_index("s")
    wid = cid * NS + sid
    n_acc = n_rows + DUMP_PAD
    acc_slab = n_acc // NS
    pltpu.sync_copy(zeros_hbm.at[pl.ds(sid * acc_slab, acc_slab)],
                    acc.at[pl.ds(sid * acc_slab, acc_slab)])
    plsc.subcore_barrier()

    for h in range(2):
        pltpu.sync_copy(src_hbm.at[2 * wid + h], src_v)
        pltpu.sync_copy(dst_hbm.at[2 * wid + h], dst_v)

        def rnd(r, carry):
            j0 = 2 * r
            g0 = pltpu.async_copy(x_hbm.at[src_v.at[j0]], rows0, gsem0)
            g1 = pltpu.async_copy(x_hbm.at[src_v.at[j0 + 1]], rows1, gsem1)
            g0.wait()
            pltpu.sync_copy(rows0, acc.at[dst_v.at[j0]], add=True)
            g1.wait()
            pltpu.sync_copy(rows1, acc.at[dst_v.at[j0 + 1]], add=True)
            return carry

        lax.fori_loop(0, k // 4, rnd, 0)
    plsc.subcore_barrier()
    pltpu.sync_copy(acc.at[pl.ds(sid * acc_slab, acc_slab)],
                    out_hbm.at[cid, pl.ds(sid * acc_slab, acc_slab)])


def _count_body(n_rows, k, dst_hbm, zeros_hbm, out_hbm,
                dst_v, ones_v, acc, sem):
    cid = lax.axis_index("c")
    sid = lax.axis_index("s")
    wid = cid * NS + sid
    n_acc = n_rows + DUMP_PAD
    acc_slab = n_acc // NS
    pltpu.sync_copy(dst_hbm.at[wid], dst_v)
    ones16 = jnp.ones((16,), jnp.float32)

    def obody(r, carry):
        for g in range(D // 16):
            ones_v[r, pl.ds(g * 16, 16)] = ones16
        return carry

    lax.fori_loop(0, CH, obody, 0)
    pltpu.sync_copy(zeros_hbm.at[pl.ds(sid * acc_slab, acc_slab)],
                    acc.at[pl.ds(sid * acc_slab, acc_slab)])
    plsc.subcore_barrier()

    def rnd(r, carry):
        base = r * 8
        scatters = [
            pltpu.async_copy(ones_v, acc.at[dst_v.at[base + q]], sem,
                             add=True)
            for q in range(8)
        ]
        for s in scatters:
            s.wait()
        return carry

    lax.fori_loop(0, k // 8, rnd, 0)
    plsc.subcore_barrier()
    pltpu.sync_copy(acc.at[pl.ds(sid * acc_slab, acc_slab)],
                    out_hbm.at[cid, pl.ds(sid * acc_slab, acc_slab)])


@functools.lru_cache(maxsize=None)
def _make_seg_sum(n_rows, k):
    return pl.kernel(
        functools.partial(_seg_sum_body, n_rows, k),
        out_type=jax.ShapeDtypeStruct((NC, n_rows + DUMP_PAD, D), jnp.float32),
        mesh=_mesh,
        scratch_types=[
            pltpu.VMEM((k // 2, CH), jnp.int32),
            pltpu.VMEM((k // 2, CH), jnp.int32),
            pltpu.VMEM((CH, D), jnp.float32),
            pltpu.VMEM((CH, D), jnp.float32),
            pltpu.VMEM_SHARED((n_rows + DUMP_PAD, D), jnp.float32),
            pltpu.SemaphoreType.DMA,
            pltpu.SemaphoreType.DMA,
        ],
    )


@functools.lru_cache(maxsize=None)
def _make_count(n_rows, k):
    return pl.kernel(
        functools.partial(_count_body, n_rows, k),
        out_type=jax.ShapeDtypeStruct((NC, n_rows + DUMP_PAD, D), jnp.float32),
        mesh=_mesh,
        scratch_types=[
            pltpu.VMEM((k, CH), jnp.int32),
            pltpu.VMEM((CH, D), jnp.float32),
            pltpu.VMEM_SHARED((n_rows + DUMP_PAD, D), jnp.float32),
            pltpu.SemaphoreType.DMA,
        ],
    )


def _tc_update_body(agg_ref, cnt_ref, h_ref, w1_ref, w2_ref, o_ref):
    a = agg_ref[0] + agg_ref[1]
    c = cnt_ref[0][:, 0:1] + cnt_ref[1][:, 0:1]
    a = a / jnp.maximum(c, 1.0)
    o_ref[...] = jnp.maximum(
        jnp.dot(a, w1_ref[...], preferred_element_type=jnp.float32,
                precision=lax.Precision.HIGHEST)
        + jnp.dot(h_ref[...], w2_ref[...], preferred_element_type=jnp.float32,
                  precision=lax.Precision.HIGHEST),
        0.0,
    )


def _tc_update(agg, cnt, h, w1, w2):
    n = h.shape[0]
    b = 1000
    return pl.pallas_call(
        _tc_update_body,
        grid=(n // b,),
        in_specs=[
            pl.BlockSpec((NC, b, D), lambda i: (0, i, 0)),
            pl.BlockSpec((NC, b, D), lambda i: (0, i, 0)),
            pl.BlockSpec((b, D), lambda i: (i, 0)),
            pl.BlockSpec((D, D), lambda i: (0, 0)),
            pl.BlockSpec((D, D), lambda i: (0, 0)),
        ],
        out_specs=pl.BlockSpec((b, D), lambda i: (i, 0)),
        out_shape=jax.ShapeDtypeStruct((n, D), jnp.float32),
    )(agg, cnt, h, w1, w2)


def _tc_tanh_body(x_ref, w_ref, o_ref):
    o_ref[...] = jnp.tanh(
        jnp.dot(x_ref[...], w_ref[...], preferred_element_type=jnp.float32,
                precision=lax.Precision.HIGHEST))


def _tc_tanh(x, w):
    n = x.shape[0]
    b = 1000
    return pl.pallas_call(
        _tc_tanh_body,
        grid=(n // b,),
        in_specs=[
            pl.BlockSpec((b, D), lambda i: (i, 0)),
            pl.BlockSpec((D, D), lambda i: (0, 0)),
        ],
        out_specs=pl.BlockSpec((b, D), lambda i: (i, 0)),
        out_shape=jax.ShapeDtypeStruct((n, D), jnp.float32),
    )(x, w)


def kernel(d_feat, p_feat, dd_edge_index, dp_edge_index,
           Wd_att, Wp_att, W_dd, W_dd_self, W_dp, W_p_self, W_pd, W_d_self):
    n_drug = d_feat.shape[0]
    n_prot = p_feat.shape[0]
    e = dd_edge_index.shape[1]
    k = -(-e // (NW * CH))
    k = -(-k // 4) * 4
    e_pad = NW * k * CH

    def prep(idx, fill, spread=1):
        idx = idx.astype(jnp.int32)
        pad = fill + jnp.arange(e_pad - e, dtype=jnp.int32) % spread
        return jnp.concatenate([idx, pad]).reshape(NW, k, CH)

    def vk(a):
        return a

    def v3(a):
        return a.reshape(NW * 2, k // 2, CH)

    src_dd_g = prep(dd_edge_index[0], 0, n_drug)
    dst_dd_s = prep(dd_edge_index[1], n_drug, DUMP_PAD)
    src_dp_g = prep(dp_edge_index[0], 0, n_drug)
    src_dp_s = prep(dp_edge_index[0], n_drug, DUMP_PAD)
    dst_dp_g = prep(dp_edge_index[1], 0, n_prot)
    dst_dp_s = prep(dp_edge_index[1], n_prot, DUMP_PAD)

    zeros_acc = jnp.zeros((n_drug + DUMP_PAD, D), jnp.float32)

    seg_sum = _make_seg_sum(n_drug, k)
    count = _make_count(n_drug, k)

    cnt_dd = count(vk(dst_dd_s), zeros_acc)
    cnt_p = count(vk(dst_dp_s), zeros_acc)
    cnt_d = count(vk(src_dp_s), zeros_acc)

    d_att = _tc_tanh(d_feat, Wd_att)
    p_att = _tc_tanh(p_feat, Wp_att)
    d, p = d_att, p_att
    for _ in range(2):
        s_dd = seg_sum(d, v3(src_dd_g), v3(dst_dd_s), zeros_acc)
        d = _tc_update(s_dd, cnt_dd, d, W_dd, W_dd_self)
        s_pd = seg_sum(p, v3(dst_dp_g), v3(src_dp_s), zeros_acc)
        s_dp = seg_sum(d, v3(src_dp_g), v3(dst_dp_s), zeros_acc)
        p_new = _tc_update(s_dp, cnt_p, p, W_dp, W_p_self)
        d = _tc_update(s_pd, cnt_d, d, W_pd, W_d_self)
        p = p_new
    return jnp.concatenate([d, p, d_att, p_att], axis=0)
